# baseline (device time: 154537 ns/iter reference)
import jax
import jax.numpy as jnp
from jax import lax
from jax.experimental import pallas as pl
from jax.experimental.pallas import tpu as pltpu

EPS = 1e-5
N_GLOBAL = 4096.0
BM = 1024


def _body(x_ref, g_ref, o_ref, acc_ref, inv_ref, comm_ref, send_sem, recv_sem):
    i = pl.program_id(0)
    nb = pl.num_programs(0) // 2

    @pl.when(i < nb)
    def _phase1():
        x = x_ref[...]
        acc_ref[pl.ds(i * BM, BM), :] = jnp.sum(x * x, axis=1, keepdims=True)

    @pl.when(i == nb - 1)
    def _exchange():
        my_x = lax.axis_index("x")
        my_y = lax.axis_index("y")
        peer = (my_x, 1 - my_y)

        barrier_sem = pltpu.get_barrier_semaphore()
        pl.semaphore_signal(
            barrier_sem, inc=1, device_id=peer,
            device_id_type=pl.DeviceIdType.MESH,
        )
        pl.semaphore_wait(barrier_sem, 1)

        rdma = pltpu.make_async_remote_copy(
            src_ref=acc_ref,
            dst_ref=comm_ref,
            send_sem=send_sem,
            recv_sem=recv_sem,
            device_id=peer,
            device_id_type=pl.DeviceIdType.MESH,
        )
        rdma.start()
        rdma.wait()

        total = acc_ref[...] + comm_ref[...]
        inv_ref[...] = lax.rsqrt(total * (1.0 / N_GLOBAL) + EPS)

    @pl.when(i >= nb)
    def _phase2():
        j = i - nb
        r = inv_ref[pl.ds(j * BM, BM), :]
        o_ref[...] = x_ref[...] * r * g_ref[...]


def kernel(x, gamma):
    m, n = x.shape
    nb = m // BM

    return pl.pallas_call(
        _body,
        grid=(2 * nb,),
        in_specs=[
            pl.BlockSpec((BM, n), lambda i: (jnp.where(i < nb, i, i - nb), 0)),
            pl.BlockSpec((n,), lambda i: (0,)),
        ],
        out_specs=pl.BlockSpec((BM, n), lambda i: (jnp.where(i < nb, 0, i - nb), 0)),
        out_shape=jax.ShapeDtypeStruct((m, n), jnp.float32),
        scratch_shapes=[
            pltpu.VMEM((m, 1), jnp.float32),
            pltpu.VMEM((m, 1), jnp.float32),
            pltpu.VMEM((m, 1), jnp.float32),
            pltpu.SemaphoreType.DMA,
            pltpu.SemaphoreType.DMA,
        ],
        compiler_params=pltpu.CompilerParams(
            collective_id=0,
            vmem_limit_bytes=56 * 1024 * 1024,
        ),
    )(x, gamma)
